# baseline (speedup 1.0000x reference)
import jax
import jax.numpy as jnp
from jax.experimental import pallas as pl
from jax.experimental.pallas import tpu as pltpu


def _make_hamilton(w):
    r, i, j, k = jnp.split(w, 4, axis=1)
    r2 = jnp.concatenate([r, -i, -j, -k], axis=0)
    i2 = jnp.concatenate([i, r, -k, j], axis=0)
    j2 = jnp.concatenate([j, k, r, -i], axis=0)
    k2 = jnp.concatenate([k, -j, i, r], axis=0)
    return jnp.concatenate([r2, i2, j2, k2], axis=1)


def _fused_kernel(x_ref, adj_ref, w_ref, g_ref, b_ref, y_ref,
                  sup_ref, s1_ref, s2_ref, *, rb, nr, n):
    i = pl.program_id(0)

    @pl.when(i == 0)
    def _():
        hamilton = _make_hamilton(w_ref[...])
        sup_ref[...] = jnp.dot(x_ref[...], hamilton,
                               preferred_element_type=jnp.float32)
        s1_ref[...] = jnp.zeros_like(s1_ref)
        s2_ref[...] = jnp.zeros_like(s2_ref)

    o = jnp.dot(adj_ref[...], sup_ref[...], preferred_element_type=jnp.float32)
    y_ref[pl.ds(i * rb, rb), :] = o
    s1_ref[...] += jnp.sum(o, axis=0, keepdims=True)
    s2_ref[...] += jnp.sum(o * o, axis=0, keepdims=True)

    @pl.when(i == nr - 1)
    def _():
        mean = s1_ref[...] / n
        var = s2_ref[...] / n - mean * mean
        inv = jax.lax.rsqrt(var + 1e-5)
        scale = inv * g_ref[...]
        shift = b_ref[...] - mean * scale
        y_ref[...] = jnp.tanh(y_ref[...] * scale + shift)


def kernel(x, adj, weight, gamma, beta):
    n, _ = x.shape
    fout = weight.shape[1]
    rb = 80
    nr = n // rb

    import functools
    y = pl.pallas_call(
        functools.partial(_fused_kernel, rb=rb, nr=nr, n=float(n)),
        grid=(nr,),
        in_specs=[
            pl.BlockSpec((n, x.shape[1]), lambda i: (0, 0)),
            pl.BlockSpec((rb, n), lambda i: (i, 0)),
            pl.BlockSpec(weight.shape, lambda i: (0, 0)),
            pl.BlockSpec((1, fout), lambda i: (0, 0)),
            pl.BlockSpec((1, fout), lambda i: (0, 0)),
        ],
        out_specs=pl.BlockSpec((n, fout), lambda i: (0, 0)),
        out_shape=jax.ShapeDtypeStruct((n, fout), jnp.float32),
        scratch_shapes=[
            pltpu.VMEM((n, fout), jnp.float32),
            pltpu.VMEM((1, fout), jnp.float32),
            pltpu.VMEM((1, fout), jnp.float32),
        ],
    )(x, adj, weight, gamma.reshape(1, fout), beta.reshape(1, fout))
    return y


# speedup vs baseline: 1.3315x; 1.3315x over previous
import jax
import jax.numpy as jnp
from jax.experimental import pallas as pl
from jax.experimental.pallas import tpu as pltpu


def _make_hamilton(w):
    r, i, j, k = jnp.split(w, 4, axis=1)
    r2 = jnp.concatenate([r, -i, -j, -k], axis=0)
    i2 = jnp.concatenate([i, r, -k, j], axis=0)
    j2 = jnp.concatenate([j, k, r, -i], axis=0)
    k2 = jnp.concatenate([k, -j, i, r], axis=0)
    return jnp.concatenate([r2, i2, j2, k2], axis=1)


def _fused_kernel(x_ref, adja_ref, adjb_ref, w_ref, g_ref, b_ref, y_ref,
                  sup_ref, s1_ref, s2_ref, *, rb, nr, n):
    i = pl.program_id(0)

    @pl.when(i == 0)
    def _():
        hamilton = _make_hamilton(w_ref[...])
        sup_ref[...] = jnp.dot(x_ref[...], hamilton,
                               preferred_element_type=jnp.float32)
        s1_ref[...] = jnp.zeros_like(s1_ref)
        s2_ref[...] = jnp.zeros_like(s2_ref)

    oa = jnp.dot(adja_ref[...], sup_ref[...], preferred_element_type=jnp.float32)
    y_ref[pl.ds((2 * i) * rb, rb), :] = oa
    ob = jnp.dot(adjb_ref[...], sup_ref[...], preferred_element_type=jnp.float32)
    y_ref[pl.ds((2 * i + 1) * rb, rb), :] = ob
    s1_ref[...] += (jnp.sum(oa, axis=0, keepdims=True)
                    + jnp.sum(ob, axis=0, keepdims=True))
    s2_ref[...] += (jnp.sum(oa * oa, axis=0, keepdims=True)
                    + jnp.sum(ob * ob, axis=0, keepdims=True))

    @pl.when(i == nr - 1)
    def _():
        mean = s1_ref[...] / n
        var = s2_ref[...] / n - mean * mean
        inv = jax.lax.rsqrt(var + 1e-5)
        scale = inv * g_ref[...]
        shift = b_ref[...] - mean * scale
        y_ref[...] = jnp.tanh(y_ref[...] * scale + shift)


def kernel(x, adj, weight, gamma, beta):
    n, _ = x.shape
    fout = weight.shape[1]
    rb = 200
    nr = n // (2 * rb)

    import functools
    y = pl.pallas_call(
        functools.partial(_fused_kernel, rb=rb, nr=nr, n=float(n)),
        grid=(nr,),
        in_specs=[
            pl.BlockSpec((n, x.shape[1]), lambda i: (0, 0)),
            pl.BlockSpec((rb, n), lambda i: (2 * i, 0)),
            pl.BlockSpec((rb, n), lambda i: (2 * i + 1, 0)),
            pl.BlockSpec(weight.shape, lambda i: (0, 0)),
            pl.BlockSpec((1, fout), lambda i: (0, 0)),
            pl.BlockSpec((1, fout), lambda i: (0, 0)),
        ],
        out_specs=pl.BlockSpec((n, fout), lambda i: (0, 0)),
        out_shape=jax.ShapeDtypeStruct((n, fout), jnp.float32),
        scratch_shapes=[
            pltpu.VMEM((n, fout), jnp.float32),
            pltpu.VMEM((1, fout), jnp.float32),
            pltpu.VMEM((1, fout), jnp.float32),
        ],
    )(x, adj, adj, weight, gamma.reshape(1, fout), beta.reshape(1, fout))
    return y


# speedup vs baseline: 1.3617x; 1.0227x over previous
import functools

import jax
import jax.numpy as jnp
from jax.experimental import pallas as pl
from jax.experimental.pallas import tpu as pltpu


def _make_hamilton(w):
    r, i, j, k = jnp.split(w, 4, axis=1)
    r2 = jnp.concatenate([r, -i, -j, -k], axis=0)
    i2 = jnp.concatenate([i, r, -k, j], axis=0)
    j2 = jnp.concatenate([j, k, r, -i], axis=0)
    k2 = jnp.concatenate([k, -j, i, r], axis=0)
    return jnp.concatenate([r2, i2, j2, k2], axis=1)


def _fused_kernel(x_ref, adj_ref, w_ref, g_ref, b_ref, y_ref,
                  sup_ref, out_ref, s1_ref, s2_ref, sem, *, rb, nr, n, nchunks):
    i = pl.program_id(0)

    @pl.when(i == 0)
    def _():
        hamilton = _make_hamilton(w_ref[...])
        sup_ref[...] = jnp.dot(x_ref[...], hamilton,
                               preferred_element_type=jnp.float32)
        s1_ref[...] = jnp.zeros_like(s1_ref)
        s2_ref[...] = jnp.zeros_like(s2_ref)

    o = jnp.dot(adj_ref[...], sup_ref[...], preferred_element_type=jnp.float32)
    out_ref[pl.ds(i * rb, rb), :] = o
    s1_ref[...] += jnp.sum(o, axis=0, keepdims=True)
    s2_ref[...] += jnp.sum(o * o, axis=0, keepdims=True)

    @pl.when(i == nr - 1)
    def _():
        mean = s1_ref[...] / n
        var = s2_ref[...] / n - mean * mean
        inv = jax.lax.rsqrt(var + 1e-5)
        scale = inv * g_ref[...]
        shift = b_ref[...] - mean * scale
        cb = out_ref.shape[0] // nchunks
        for c in range(nchunks):
            rows = pl.ds(c * cb, cb)
            out_ref[rows, :] = jnp.tanh(out_ref[rows, :] * scale + shift)
            pltpu.make_async_copy(out_ref.at[rows, :], y_ref.at[rows, :],
                                  sem.at[c]).start()
        for c in range(nchunks):
            rows = pl.ds(c * cb, cb)
            pltpu.make_async_copy(out_ref.at[rows, :], y_ref.at[rows, :],
                                  sem.at[c]).wait()


def kernel(x, adj, weight, gamma, beta):
    n, _ = x.shape
    fout = weight.shape[1]
    rb = 400
    nr = n // rb
    nchunks = 5

    y = pl.pallas_call(
        functools.partial(_fused_kernel, rb=rb, nr=nr, n=float(n),
                          nchunks=nchunks),
        grid=(nr,),
        in_specs=[
            pl.BlockSpec((n, x.shape[1]), lambda i: (0, 0)),
            pl.BlockSpec((rb, n), lambda i: (i, 0)),
            pl.BlockSpec(weight.shape, lambda i: (0, 0)),
            pl.BlockSpec((1, fout), lambda i: (0, 0)),
            pl.BlockSpec((1, fout), lambda i: (0, 0)),
        ],
        out_specs=pl.BlockSpec(memory_space=pltpu.MemorySpace.HBM),
        out_shape=jax.ShapeDtypeStruct((n, fout), jnp.float32),
        scratch_shapes=[
            pltpu.VMEM((n, fout), jnp.float32),
            pltpu.VMEM((n, fout), jnp.float32),
            pltpu.VMEM((1, fout), jnp.float32),
            pltpu.VMEM((1, fout), jnp.float32),
            pltpu.SemaphoreType.DMA((nchunks,)),
        ],
    )(x, adj, weight, gamma.reshape(1, fout), beta.reshape(1, fout))
    return y
